# baseline (speedup 1.0000x reference)
import functools

import jax
import jax.numpy as jnp
from jax import lax
from jax.experimental import pallas as pl
from jax.experimental.pallas import tpu as pltpu
from jax.experimental.pallas import tpu_sc as plsc

N = 10000
E = 320000
D = 128

NC = 2
NS = 16
NW = NC * NS
CHUNK = 128
NCHUNK = 80
HB = NCHUNK // 2
EPT = NCHUNK * CHUNK
EPAD = EPT * NW
NP = 10112
RPT = NP // NS
BLK = 2000
BN_SCALE = float((1.0 + 1e-5) ** -0.5)

_mesh = plsc.VectorSubcoreMesh(core_axis_name="c", subcore_axis_name="s")


@functools.partial(
    pl.kernel,
    mesh=_mesh,
    out_type=jax.ShapeDtypeStruct((NC, NP, D), jnp.float32),
    scratch_types=[
        pltpu.VMEM((NCHUNK, CHUNK), jnp.int32),
        pltpu.VMEM((CHUNK, D), jnp.float32),
        pltpu.VMEM((CHUNK, D), jnp.float32),
        pltpu.VMEM_SHARED((NP, D), jnp.float32),
        pltpu.SemaphoreType.DMA,
        pltpu.SemaphoreType.DMA,
    ],
)
def _sc_degree(dstw_hbm, ones_hbm, zeros_hbm, out_hbm, dst_v, ones_v, ones2_v,
               deg_sh, semA, semB):
    c = lax.axis_index("c")
    s = lax.axis_index("s")
    w = s * NC + c
    r0 = s * RPT
    pltpu.sync_copy(zeros_hbm.at[pl.ds(r0, RPT)], deg_sh.at[pl.ds(r0, RPT)])
    pltpu.sync_copy(dstw_hbm.at[w], dst_v)
    pltpu.sync_copy(ones_hbm, ones_v)
    pltpu.sync_copy(ones_hbm, ones2_v)
    plsc.subcore_barrier()

    pltpu.async_copy(ones_v, deg_sh.at[dst_v.at[0]], semA, add=True)
    pltpu.async_copy(ones2_v, deg_sh.at[dst_v.at[1]], semB, add=True)

    def body(i, carry):
        t2 = 2 * i
        pltpu.async_copy(ones_v, deg_sh.at[dst_v.at[t2 + 2]], semA, add=True)
        pltpu.make_async_copy(ones_v, deg_sh.at[dst_v.at[t2]], semA).wait()
        pltpu.async_copy(ones2_v, deg_sh.at[dst_v.at[t2 + 3]], semB, add=True)
        pltpu.make_async_copy(ones2_v, deg_sh.at[dst_v.at[t2 + 1]], semB).wait()
        return carry

    lax.fori_loop(0, NCHUNK // 2 - 1, body, 0)
    pltpu.make_async_copy(ones_v, deg_sh.at[dst_v.at[NCHUNK - 2]], semA).wait()
    pltpu.make_async_copy(ones2_v, deg_sh.at[dst_v.at[NCHUNK - 1]], semB).wait()
    plsc.subcore_barrier()
    pltpu.sync_copy(deg_sh.at[pl.ds(r0, RPT)], out_hbm.at[c, pl.ds(r0, RPT)])


@functools.partial(
    pl.kernel,
    mesh=_mesh,
    out_type=jax.ShapeDtypeStruct((NC, NP, D), jnp.float32),
    scratch_types=[
        pltpu.VMEM((HB + 8, CHUNK), jnp.int32),
        pltpu.VMEM((HB, CHUNK), jnp.int32),
        pltpu.VMEM((CHUNK, D), jnp.float32),
        pltpu.VMEM((CHUNK, D), jnp.float32),
        pltpu.VMEM_SHARED((NP, D), jnp.float32),
        pltpu.SemaphoreType.DMA,
        pltpu.SemaphoreType.DMA,
    ],
)
def _sc_segsum(h_hbm, srcw_hbm, dstw_hbm, zeros_hbm, out_hbm,
               src_v, dst_v, buf0, buf1, acc_sh, sem0, sem1):
    c = lax.axis_index("c")
    s = lax.axis_index("s")
    w = s * NC + c
    r0 = s * RPT
    pltpu.sync_copy(srcw_hbm.at[w, pl.ds(0, HB + 8)], src_v)
    pltpu.sync_copy(dstw_hbm.at[w, pl.ds(0, HB)], dst_v)
    pltpu.async_copy(h_hbm.at[src_v.at[0]], buf0, sem0)
    pltpu.sync_copy(zeros_hbm.at[pl.ds(r0, RPT)], acc_sh.at[pl.ds(r0, RPT)])
    plsc.subcore_barrier()

    for half in range(2):
        if half == 1:
            pltpu.sync_copy(srcw_hbm.at[w, pl.ds(HB, HB + 8)], src_v)
            pltpu.sync_copy(dstw_hbm.at[w, pl.ds(HB, HB)], dst_v)
            pltpu.async_copy(h_hbm.at[src_v.at[0]], buf0, sem0)

        def body(i, carry):
            t = 2 * i
            pltpu.async_copy(h_hbm.at[src_v.at[t + 1]], buf1, sem1)
            pltpu.make_async_copy(h_hbm.at[src_v.at[t]], buf0, sem0).wait()
            pltpu.sync_copy(buf0, acc_sh.at[dst_v.at[t]], add=True)
            pltpu.async_copy(h_hbm.at[src_v.at[t + 2]], buf0, sem0)
            pltpu.make_async_copy(h_hbm.at[src_v.at[t + 1]], buf1, sem1).wait()
            pltpu.sync_copy(buf1, acc_sh.at[dst_v.at[t + 1]], add=True)
            return carry

        lax.fori_loop(0, HB // 2, body, 0)
        pltpu.make_async_copy(h_hbm.at[src_v.at[HB]], buf0, sem0).wait()
    plsc.subcore_barrier()
    pltpu.sync_copy(acc_sh.at[pl.ds(r0, RPT)], out_hbm.at[c, pl.ds(r0, RPT)])


def _row_spec(width):
    return pl.BlockSpec((BLK, width), lambda i: (i, 0))


_full = lambda shape: pl.BlockSpec(shape, lambda i: tuple(0 for _ in shape))
_acc_spec = pl.BlockSpec((NC, BLK, D), lambda i: (0, i, 0))


def _tc_mm_body(x_ref, w_ref, out_ref):
    out_ref[...] = jnp.dot(x_ref[...], w_ref[...],
                           preferred_element_type=jnp.float32)


_tc_mm = pl.pallas_call(
    _tc_mm_body,
    grid=(N // BLK,),
    in_specs=[_row_spec(D), _full((D, D))],
    out_specs=_row_spec(D),
    out_shape=jax.ShapeDtypeStruct((N, D), jnp.float32),
)


def _tc_scale_body(g_ref, degp_ref, hp_ref, dinv_ref):
    deg = degp_ref[0, :, 0:1] + degp_ref[1, :, 0:1] + 1.0
    dinv = lax.rsqrt(deg)
    hp_ref[...] = dinv * g_ref[...]
    dinv_ref[...] = jnp.broadcast_to(dinv, (BLK, 8))


_tc_scale = pl.pallas_call(
    _tc_scale_body,
    grid=(N // BLK,),
    in_specs=[_row_spec(D), _acc_spec],
    out_specs=[_row_spec(D), _row_spec(8)],
    out_shape=[jax.ShapeDtypeStruct((N, D), jnp.float32),
               jax.ShapeDtypeStruct((N, 8), jnp.float32)],
)


def _tc_mid_body(accp_ref, hp_ref, dinv_ref, b_ref, g_ref, be_ref, w_ref, out_ref):
    dinv = dinv_ref[:, 0:1]
    z = dinv * (accp_ref[0] + accp_ref[1] + hp_ref[...]) + b_ref[...]
    zb = g_ref[...] * (z * BN_SCALE) + be_ref[...]
    y = zb * jax.nn.sigmoid(zb)
    out_ref[...] = dinv * jnp.dot(y, w_ref[...],
                                  preferred_element_type=jnp.float32)


_tc_mid = pl.pallas_call(
    _tc_mid_body,
    grid=(N // BLK,),
    in_specs=[_acc_spec, _row_spec(D), _row_spec(8),
              _full((1, D)), _full((1, D)), _full((1, D)), _full((D, D))],
    out_specs=_row_spec(D),
    out_shape=jax.ShapeDtypeStruct((N, D), jnp.float32),
)


def _tc_fin_body(accp_ref, hp_ref, dinv_ref, b_ref, out_ref):
    out_ref[...] = dinv_ref[:, 0:1] * (accp_ref[0] + accp_ref[1] + hp_ref[...]) + b_ref[...]


_tc_fin = pl.pallas_call(
    _tc_fin_body,
    grid=(N // BLK,),
    in_specs=[_acc_spec, _row_spec(D), _row_spec(8), _full((1, D))],
    out_specs=_row_spec(D),
    out_shape=jax.ShapeDtypeStruct((N, D), jnp.float32),
)


def kernel(x, adj_t, W1, b1, g1, be1, W2, b2, g2, be2, W3, b3):
    src = adj_t[0].astype(jnp.int32)
    dst = adj_t[1].astype(jnp.int32)
    npad = EPAD - E
    ar = jnp.arange(npad, dtype=jnp.int32)
    pad_src = (ar * 97) % N
    pad_dst = N + (ar % (NP - N))
    srcp = jnp.concatenate([src, pad_src]).reshape(NW, NCHUNK, CHUNK)
    dstp = jnp.concatenate([dst, pad_dst]).reshape(NW, NCHUNK, CHUNK)
    extra = jnp.broadcast_to(
        ((jnp.arange(CHUNK, dtype=jnp.int32) * 131) % N)[None, None, :],
        (NW, 8, CHUNK))
    srcp = jnp.concatenate([srcp, extra], axis=1)

    zerosD = jnp.zeros((NP, D), jnp.float32)
    ones = jnp.ones((CHUNK, D), jnp.float32)
    b1r, b2r, b3r = b1.reshape(1, D), b2.reshape(1, D), b3.reshape(1, D)
    g1r, g2r = g1.reshape(1, D), g2.reshape(1, D)
    be1r, be2r = be1.reshape(1, D), be2.reshape(1, D)

    degp = _sc_degree(dstp, ones, zerosD)
    gg1 = _tc_mm(x, W1)
    dep = (degp[0, 0, 0] * 0.0).astype(jnp.int32)
    srcp = srcp + dep
    h1p, dinvb = _tc_scale(gg1, degp)
    acc1 = _sc_segsum(h1p, srcp, dstp, zerosD)
    h2p = _tc_mid(acc1, h1p, dinvb, b1r, g1r, be1r, W2)
    acc2 = _sc_segsum(h2p, srcp, dstp, zerosD)
    h3p = _tc_mid(acc2, h2p, dinvb, b2r, g2r, be2r, W3)
    acc3 = _sc_segsum(h3p, srcp, dstp, zerosD)
    return _tc_fin(acc3, h3p, dinvb, b3r)


# speedup vs baseline: 1.0057x; 1.0057x over previous
import functools

import jax
import jax.numpy as jnp
from jax import lax
from jax.experimental import pallas as pl
from jax.experimental.pallas import tpu as pltpu
from jax.experimental.pallas import tpu_sc as plsc

N = 10000
E = 320000
D = 128

NC = 2
NS = 16
NW = NC * NS
CHUNK = 128
NCHUNK = 80
HB = NCHUNK // 2
EPT = NCHUNK * CHUNK
EPAD = EPT * NW
NP = 10112
RPT = NP // NS
BLK = 2000
BN_SCALE = float((1.0 + 1e-5) ** -0.5)

_mesh = plsc.VectorSubcoreMesh(core_axis_name="c", subcore_axis_name="s")


@functools.partial(
    pl.kernel,
    mesh=_mesh,
    out_type=jax.ShapeDtypeStruct((NC, NP, D), jnp.float32),
    scratch_types=[
        pltpu.VMEM((NCHUNK, CHUNK), jnp.int32),
        pltpu.VMEM((CHUNK, D), jnp.float32),
        pltpu.VMEM_SHARED((NP, D), jnp.float32),
        pltpu.SemaphoreType.DMA,
    ],
)
def _sc_degree(dstw_hbm, ones_hbm, zeros_hbm, out_hbm, dst_v, ones_v, deg_sh, semA):
    c = lax.axis_index("c")
    s = lax.axis_index("s")
    w = s * NC + c
    r0 = s * RPT
    pltpu.sync_copy(zeros_hbm.at[pl.ds(r0, RPT)], deg_sh.at[pl.ds(r0, RPT)])
    pltpu.sync_copy(dstw_hbm.at[w], dst_v)
    pltpu.sync_copy(ones_hbm, ones_v)
    plsc.subcore_barrier()

    LOOKAHEAD = 8
    for j in range(LOOKAHEAD):
        pltpu.async_copy(ones_v, deg_sh.at[dst_v.at[j]], semA, add=True)

    def body(j, carry):
        pltpu.async_copy(ones_v, deg_sh.at[dst_v.at[j + LOOKAHEAD]], semA, add=True)
        pltpu.make_async_copy(ones_v, deg_sh.at[dst_v.at[j]], semA).wait()
        return carry

    lax.fori_loop(0, NCHUNK - LOOKAHEAD, body, 0)
    for j in range(NCHUNK - LOOKAHEAD, NCHUNK):
        pltpu.make_async_copy(ones_v, deg_sh.at[dst_v.at[j]], semA).wait()
    plsc.subcore_barrier()
    pltpu.sync_copy(deg_sh.at[pl.ds(r0, RPT)], out_hbm.at[c, pl.ds(r0, RPT)])


@functools.partial(
    pl.kernel,
    mesh=_mesh,
    out_type=jax.ShapeDtypeStruct((NC, NP, D), jnp.float32),
    scratch_types=[
        pltpu.VMEM((HB + 8, CHUNK), jnp.int32),
        pltpu.VMEM((HB, CHUNK), jnp.int32),
        pltpu.VMEM((CHUNK, D), jnp.float32),
        pltpu.VMEM((CHUNK, D), jnp.float32),
        pltpu.VMEM_SHARED((NP, D), jnp.float32),
        pltpu.SemaphoreType.DMA,
        pltpu.SemaphoreType.DMA,
    ],
)
def _sc_segsum(h_hbm, srcw_hbm, dstw_hbm, zeros_hbm, out_hbm,
               src_v, dst_v, buf0, buf1, acc_sh, sem0, sem1):
    c = lax.axis_index("c")
    s = lax.axis_index("s")
    w = s * NC + c
    r0 = s * RPT
    pltpu.sync_copy(srcw_hbm.at[w, pl.ds(0, HB + 8)], src_v)
    pltpu.sync_copy(dstw_hbm.at[w, pl.ds(0, HB)], dst_v)
    pltpu.async_copy(h_hbm.at[src_v.at[0]], buf0, sem0)
    pltpu.sync_copy(zeros_hbm.at[pl.ds(r0, RPT)], acc_sh.at[pl.ds(r0, RPT)])
    plsc.subcore_barrier()

    for half in range(2):
        if half == 1:
            pltpu.sync_copy(srcw_hbm.at[w, pl.ds(HB, HB + 8)], src_v)
            pltpu.sync_copy(dstw_hbm.at[w, pl.ds(HB, HB)], dst_v)
            pltpu.async_copy(h_hbm.at[src_v.at[0]], buf0, sem0)

        def body(i, carry):
            t = 2 * i
            pltpu.async_copy(h_hbm.at[src_v.at[t + 1]], buf1, sem1)
            pltpu.make_async_copy(h_hbm.at[src_v.at[t]], buf0, sem0).wait()
            pltpu.sync_copy(buf0, acc_sh.at[dst_v.at[t]], add=True)
            pltpu.async_copy(h_hbm.at[src_v.at[t + 2]], buf0, sem0)
            pltpu.make_async_copy(h_hbm.at[src_v.at[t + 1]], buf1, sem1).wait()
            pltpu.sync_copy(buf1, acc_sh.at[dst_v.at[t + 1]], add=True)
            return carry

        lax.fori_loop(0, HB // 2, body, 0)
        pltpu.make_async_copy(h_hbm.at[src_v.at[HB]], buf0, sem0).wait()
    plsc.subcore_barrier()
    pltpu.sync_copy(acc_sh.at[pl.ds(r0, RPT)], out_hbm.at[c, pl.ds(r0, RPT)])


def _row_spec(width):
    return pl.BlockSpec((BLK, width), lambda i: (i, 0))


_full = lambda shape: pl.BlockSpec(shape, lambda i: tuple(0 for _ in shape))
_acc_spec = pl.BlockSpec((NC, BLK, D), lambda i: (0, i, 0))


def _tc_mm_body(x_ref, w_ref, out_ref):
    out_ref[...] = jnp.dot(x_ref[...], w_ref[...],
                           preferred_element_type=jnp.float32)


_tc_mm = pl.pallas_call(
    _tc_mm_body,
    grid=(N // BLK,),
    in_specs=[_row_spec(D), _full((D, D))],
    out_specs=_row_spec(D),
    out_shape=jax.ShapeDtypeStruct((N, D), jnp.float32),
)


def _tc_scale_body(g_ref, degp_ref, hp_ref, dinv_ref):
    deg = degp_ref[0, :, 0:1] + degp_ref[1, :, 0:1] + 1.0
    dinv = lax.rsqrt(deg)
    hp_ref[...] = dinv * g_ref[...]
    dinv_ref[...] = jnp.broadcast_to(dinv, (BLK, 8))


_tc_scale = pl.pallas_call(
    _tc_scale_body,
    grid=(N // BLK,),
    in_specs=[_row_spec(D), _acc_spec],
    out_specs=[_row_spec(D), _row_spec(8)],
    out_shape=[jax.ShapeDtypeStruct((N, D), jnp.float32),
               jax.ShapeDtypeStruct((N, 8), jnp.float32)],
)


def _tc_mid_body(accp_ref, hp_ref, dinv_ref, b_ref, g_ref, be_ref, w_ref, out_ref):
    dinv = dinv_ref[:, 0:1]
    z = dinv * (accp_ref[0] + accp_ref[1] + hp_ref[...]) + b_ref[...]
    zb = g_ref[...] * (z * BN_SCALE) + be_ref[...]
    y = zb * jax.nn.sigmoid(zb)
    out_ref[...] = dinv * jnp.dot(y, w_ref[...],
                                  preferred_element_type=jnp.float32)


_tc_mid = pl.pallas_call(
    _tc_mid_body,
    grid=(N // BLK,),
    in_specs=[_acc_spec, _row_spec(D), _row_spec(8),
              _full((1, D)), _full((1, D)), _full((1, D)), _full((D, D))],
    out_specs=_row_spec(D),
    out_shape=jax.ShapeDtypeStruct((N, D), jnp.float32),
)


def _tc_fin_body(accp_ref, hp_ref, dinv_ref, b_ref, out_ref):
    out_ref[...] = dinv_ref[:, 0:1] * (accp_ref[0] + accp_ref[1] + hp_ref[...]) + b_ref[...]


_tc_fin = pl.pallas_call(
    _tc_fin_body,
    grid=(N // BLK,),
    in_specs=[_acc_spec, _row_spec(D), _row_spec(8), _full((1, D))],
    out_specs=_row_spec(D),
    out_shape=jax.ShapeDtypeStruct((N, D), jnp.float32),
)


def kernel(x, adj_t, W1, b1, g1, be1, W2, b2, g2, be2, W3, b3):
    src = adj_t[0].astype(jnp.int32)
    dst = adj_t[1].astype(jnp.int32)
    npad = EPAD - E
    ar = jnp.arange(npad, dtype=jnp.int32)
    pad_src = (ar * 97) % N
    pad_dst = N + (ar % (NP - N))
    srcp = jnp.concatenate([src, pad_src]).reshape(NW, NCHUNK, CHUNK)
    dstp = jnp.concatenate([dst, pad_dst]).reshape(NW, NCHUNK, CHUNK)
    extra = jnp.broadcast_to(
        ((jnp.arange(CHUNK, dtype=jnp.int32) * 131) % N)[None, None, :],
        (NW, 8, CHUNK))
    srcp = jnp.concatenate([srcp, extra], axis=1)

    zerosD = jnp.zeros((NP, D), jnp.float32)
    ones = jnp.ones((CHUNK, D), jnp.float32)
    b1r, b2r, b3r = b1.reshape(1, D), b2.reshape(1, D), b3.reshape(1, D)
    g1r, g2r = g1.reshape(1, D), g2.reshape(1, D)
    be1r, be2r = be1.reshape(1, D), be2.reshape(1, D)

    degp = _sc_degree(dstp, ones, zerosD)
    gg1 = _tc_mm(x, W1)
    dep = (degp[0, 0, 0] * 0.0).astype(jnp.int32)
    srcp = srcp + dep
    h1p, dinvb = _tc_scale(gg1, degp)
    acc1 = _sc_segsum(h1p, srcp, dstp, zerosD)
    h2p = _tc_mid(acc1, h1p, dinvb, b1r, g1r, be1r, W2)
    acc2 = _sc_segsum(h2p, srcp, dstp, zerosD)
    h3p = _tc_mid(acc2, h2p, dinvb, b2r, g2r, be2r, W3)
    acc3 = _sc_segsum(h3p, srcp, dstp, zerosD)
    return _tc_fin(acc3, h3p, dinvb, b3r)
